# Initial kernel scaffold; baseline (speedup 1.0000x reference)
import jax
import jax.numpy as jnp
from jax.experimental import pallas as pl


def kernel(inputs, emb):
    raise NotImplementedError("write your pallas kernel here")


# speedup vs baseline: 1.1625x; 1.1625x over previous
import functools

import jax
import jax.numpy as jnp
from jax import lax
from jax.experimental import pallas as pl
from jax.experimental.pallas import tpu as pltpu
from jax.experimental.pallas import tpu_sc as plsc

NUM_E = 8192
DIM = 256
NTOK = 8192
TM = 1024
TN = 1024
NI = NTOK // TM
NJ = NUM_E // TN
HALF = NJ // 2


def _dist_argmin_body(x_ref, e_ref, x2_ref, e2_ref, idx_ref, dsum_ref,
                      min0_ref, idx0_ref, min1_ref, idx1_ref, acc_ref):
    i = pl.program_id(0)
    j = pl.program_id(1)
    mm2 = lax.dot_general(x_ref[...], e_ref[...],
                          (((1,), (1,)), ((), ())),
                          preferred_element_type=jnp.float32)
    d = (x2_ref[...] + e2_ref[...]) - mm2
    tmin = jnp.min(d, axis=1, keepdims=True)
    col = lax.broadcasted_iota(jnp.int32, (TM, TN), 1) + j * TN
    tidx = jnp.min(jnp.where(d == tmin, col, NUM_E), axis=1, keepdims=True)

    @pl.when(j == 0)
    def _():
        min0_ref[...] = tmin
        idx0_ref[...] = tidx

    @pl.when((j > 0) & (j < HALF))
    def _():
        better = tmin < min0_ref[...]
        idx0_ref[...] = jnp.where(better, tidx, idx0_ref[...])
        min0_ref[...] = jnp.minimum(tmin, min0_ref[...])

    @pl.when(j == HALF)
    def _():
        min1_ref[...] = tmin
        idx1_ref[...] = tidx

    @pl.when(j > HALF)
    def _():
        better = tmin < min1_ref[...]
        idx1_ref[...] = jnp.where(better, tidx, idx1_ref[...])
        min1_ref[...] = jnp.minimum(tmin, min1_ref[...])

    @pl.when(j == NJ - 1)
    def _():
        m0 = min0_ref[...]
        m1 = min1_ref[...]
        m0b = m0.astype(jnp.bfloat16).astype(jnp.float32)
        use1 = m1 < m0b
        idx_ref[...] = jnp.where(use1, idx1_ref[...], idx0_ref[...])
        dchosen = jnp.where(use1, m1, m0)

        @pl.when(i == 0)
        def _():
            acc_ref[0, 0] = 0.0
        acc_ref[0, 0] += jnp.sum(dchosen)

        @pl.when(i == NI - 1)
        def _():
            dsum_ref[...] = jnp.broadcast_to(acc_ref[0, 0], (1, 1))


def _dist_argmin(flat, emb, x2, e2):
    return pl.pallas_call(
        _dist_argmin_body,
        grid=(NI, NJ),
        in_specs=[
            pl.BlockSpec((TM, DIM), lambda i, j: (i, 0)),
            pl.BlockSpec((TN, DIM), lambda i, j: (j, 0)),
            pl.BlockSpec((TM, 1), lambda i, j: (i, 0)),
            pl.BlockSpec((1, TN), lambda i, j: (0, j)),
        ],
        out_specs=[
            pl.BlockSpec((TM, 1), lambda i, j: (i, 0)),
            pl.BlockSpec((1, 1), lambda i, j: (0, 0)),
        ],
        out_shape=[
            jax.ShapeDtypeStruct((NTOK, 1), jnp.int32),
            jax.ShapeDtypeStruct((1, 1), jnp.float32),
        ],
        scratch_shapes=[
            pltpu.VMEM((TM, 1), jnp.float32),
            pltpu.VMEM((TM, 1), jnp.int32),
            pltpu.VMEM((TM, 1), jnp.float32),
            pltpu.VMEM((TM, 1), jnp.int32),
            pltpu.SMEM((1, 1), jnp.float32),
        ],
    )(flat, emb, x2, e2)


def _make_sc_gather_hist():
    info = plsc.get_sparse_core_info()
    nc, ns = info.num_cores, info.num_subcores
    nw = nc * ns
    bpw = NTOK // nw
    chunks = bpw // 128
    mesh = plsc.VectorSubcoreMesh(core_axis_name="c", subcore_axis_name="s")

    @functools.partial(
        pl.kernel, mesh=mesh,
        compiler_params=pltpu.CompilerParams(needs_layout_passes=False),
        out_type=[jax.ShapeDtypeStruct((NTOK, DIM), jnp.float32),
                  jax.ShapeDtypeStruct((nw, NUM_E), jnp.float32)],
        scratch_types=[pltpu.VMEM((chunks, 128), jnp.int32),
                       pltpu.VMEM((bpw, DIM), jnp.float32),
                       pltpu.VMEM((NUM_E,), jnp.float32),
                       pltpu.SemaphoreType.DMA],
    )
    def gather_hist(table_hbm, idx_hbm, zeros_hbm, out_hbm, hist_hbm,
                    idx_v, rows_v, hist_v, sem):
        wid = lax.axis_index("s") * nc + lax.axis_index("c")
        base = wid * bpw
        for c in range(chunks):
            pltpu.sync_copy(idx_hbm.at[wid * chunks + c], idx_v.at[c])
        for c in range(chunks):
            pltpu.async_copy(table_hbm.at[idx_v.at[c]],
                             rows_v.at[pl.ds(c * 128, 128)], sem).wait()
        pltpu.sync_copy(rows_v, out_hbm.at[pl.ds(base, bpw)])
        pltpu.sync_copy(zeros_hbm, hist_v)
        ones = jnp.full((16,), 1.0, jnp.float32)
        for c in range(chunks):
            for k in range(128 // 16):
                v = idx_v[c, pl.ds(k * 16, 16)]
                plsc.addupdate_scatter(hist_v, [v], ones)
        pltpu.sync_copy(hist_v, hist_hbm.at[wid])

    return gather_hist, nw


def _finish_body(hist_ref, dsum_ref, loss_ref, perp_ref):
    counts = jnp.sum(hist_ref[...], axis=0, keepdims=True)
    p = counts * (1.0 / NTOK)
    ent = -jnp.sum(p * jnp.log(p + 1e-10))
    perp_ref[...] = jnp.broadcast_to(jnp.exp(ent), (1, 1))
    loss_ref[...] = dsum_ref[...] * (1.25 / (NTOK * DIM))


def _finish(hist, dsum):
    return pl.pallas_call(
        _finish_body,
        out_shape=[jax.ShapeDtypeStruct((1, 1), jnp.float32),
                   jax.ShapeDtypeStruct((1, 1), jnp.float32)],
    )(hist, dsum)


def kernel(inputs, emb):
    x = jnp.transpose(inputs, (0, 2, 3, 1))
    flat = x.reshape(-1, DIM)
    x2 = jnp.sum(flat ** 2, axis=1, keepdims=True)
    e2 = jnp.sum(emb ** 2, axis=1)[None, :]
    xb = (flat * 2.0).astype(jnp.bfloat16)
    eb = emb.astype(jnp.bfloat16)
    idx2d, dsum = _dist_argmin(xb, eb, x2, e2)
    idx = idx2d.reshape(NTOK)

    gather_hist, nw = _make_sc_gather_hist()
    zeros = jnp.zeros((NUM_E,), jnp.float32)
    quant, hist = gather_hist(emb, idx.reshape(NTOK // 128, 128), zeros)

    loss2d, perp2d = _finish(hist, dsum)
    out_q = jnp.transpose(quant.reshape(8, 32, 32, DIM), (0, 3, 1, 2))
    return (loss2d[0, 0], out_q, perp2d[0, 0], idx.reshape(8, 32, 32))


# speedup vs baseline: 1.3742x; 1.1821x over previous
import functools

import jax
import jax.numpy as jnp
from jax import lax
from jax.experimental import pallas as pl
from jax.experimental.pallas import tpu as pltpu
from jax.experimental.pallas import tpu_sc as plsc

NUM_E = 8192
DIM = 256
NTOK = 8192
TM = 4096
TN = 2048
NI = NTOK // TM
NJ = NUM_E // TN
HALF = NJ // 2


def _dist_argmin_body(x_ref, e_ref, x2_ref, e2_ref, idx_ref, dsum_ref,
                      min0_ref, idx0_ref, min1_ref, idx1_ref, acc_ref):
    i = pl.program_id(0)
    j = pl.program_id(1)
    mm2 = lax.dot_general(x_ref[...], e_ref[...],
                          (((1,), (1,)), ((), ())),
                          preferred_element_type=jnp.float32)
    d = (x2_ref[...] + e2_ref[...]) - mm2
    tmin = jnp.min(d, axis=1, keepdims=True)
    col = lax.broadcasted_iota(jnp.int32, (TM, TN), 1) + j * TN
    tidx = jnp.min(jnp.where(d == tmin, col, NUM_E), axis=1, keepdims=True)

    @pl.when(j == 0)
    def _():
        min0_ref[...] = tmin
        idx0_ref[...] = tidx

    @pl.when((j > 0) & (j < HALF))
    def _():
        better = tmin < min0_ref[...]
        idx0_ref[...] = jnp.where(better, tidx, idx0_ref[...])
        min0_ref[...] = jnp.minimum(tmin, min0_ref[...])

    @pl.when(j == HALF)
    def _():
        min1_ref[...] = tmin
        idx1_ref[...] = tidx

    @pl.when(j > HALF)
    def _():
        better = tmin < min1_ref[...]
        idx1_ref[...] = jnp.where(better, tidx, idx1_ref[...])
        min1_ref[...] = jnp.minimum(tmin, min1_ref[...])

    @pl.when(j == NJ - 1)
    def _():
        m0 = min0_ref[...]
        m1 = min1_ref[...]
        m0b = m0.astype(jnp.bfloat16).astype(jnp.float32)
        use1 = m1 < m0b
        idx_ref[...] = jnp.where(use1, idx1_ref[...], idx0_ref[...])
        dchosen = jnp.where(use1, m1, m0)

        @pl.when(i == 0)
        def _():
            acc_ref[0, 0] = 0.0
        acc_ref[0, 0] += jnp.sum(dchosen)

        @pl.when(i == NI - 1)
        def _():
            dsum_ref[...] = jnp.broadcast_to(acc_ref[0, 0], (1, 1))


def _dist_argmin(flat, emb, x2, e2):
    return pl.pallas_call(
        _dist_argmin_body,
        grid=(NI, NJ),
        in_specs=[
            pl.BlockSpec((TM, DIM), lambda i, j: (i, 0)),
            pl.BlockSpec((TN, DIM), lambda i, j: (j, 0)),
            pl.BlockSpec((TM, 1), lambda i, j: (i, 0)),
            pl.BlockSpec((1, TN), lambda i, j: (0, j)),
        ],
        out_specs=[
            pl.BlockSpec((TM, 1), lambda i, j: (i, 0)),
            pl.BlockSpec((1, 1), lambda i, j: (0, 0)),
        ],
        out_shape=[
            jax.ShapeDtypeStruct((NTOK, 1), jnp.int32),
            jax.ShapeDtypeStruct((1, 1), jnp.float32),
        ],
        scratch_shapes=[
            pltpu.VMEM((TM, 1), jnp.float32),
            pltpu.VMEM((TM, 1), jnp.int32),
            pltpu.VMEM((TM, 1), jnp.float32),
            pltpu.VMEM((TM, 1), jnp.int32),
            pltpu.SMEM((1, 1), jnp.float32),
        ],
    )(flat, emb, x2, e2)


def _make_sc_gather_hist():
    info = plsc.get_sparse_core_info()
    nc, ns = info.num_cores, info.num_subcores
    nw = nc * ns
    bpw = NTOK // nw
    chunks = bpw // 128
    mesh = plsc.VectorSubcoreMesh(core_axis_name="c", subcore_axis_name="s")

    @functools.partial(
        pl.kernel, mesh=mesh,
        compiler_params=pltpu.CompilerParams(needs_layout_passes=False),
        out_type=[jax.ShapeDtypeStruct((NTOK, DIM), jnp.float32),
                  jax.ShapeDtypeStruct((nw, NUM_E), jnp.float32)],
        scratch_types=[pltpu.VMEM((chunks, 128), jnp.int32),
                       pltpu.VMEM((bpw, DIM), jnp.float32),
                       pltpu.VMEM((NUM_E,), jnp.float32),
                       pltpu.SemaphoreType.DMA],
    )
    def gather_hist(table_hbm, idx_hbm, zeros_hbm, out_hbm, hist_hbm,
                    idx_v, rows_v, hist_v, sem):
        wid = lax.axis_index("s") * nc + lax.axis_index("c")
        base = wid * bpw
        for c in range(chunks):
            pltpu.sync_copy(idx_hbm.at[wid * chunks + c], idx_v.at[c])
        for c in range(chunks):
            pltpu.async_copy(table_hbm.at[idx_v.at[c]],
                             rows_v.at[pl.ds(c * 128, 128)], sem).wait()
        pltpu.sync_copy(rows_v, out_hbm.at[pl.ds(base, bpw)])
        pltpu.sync_copy(zeros_hbm, hist_v)
        ones = jnp.full((16,), 1.0, jnp.float32)
        for c in range(chunks):
            for k in range(128 // 16):
                v = idx_v[c, pl.ds(k * 16, 16)]
                plsc.addupdate_scatter(hist_v, [v], ones)
        pltpu.sync_copy(hist_v, hist_hbm.at[wid])

    return gather_hist, nw


def _finish_body(hist_ref, dsum_ref, loss_ref, perp_ref):
    counts = jnp.sum(hist_ref[...], axis=0, keepdims=True)
    p = counts * (1.0 / NTOK)
    ent = -jnp.sum(p * jnp.log(p + 1e-10))
    perp_ref[...] = jnp.broadcast_to(jnp.exp(ent), (1, 1))
    loss_ref[...] = dsum_ref[...] * (1.25 / (NTOK * DIM))


def _finish(hist, dsum):
    return pl.pallas_call(
        _finish_body,
        out_shape=[jax.ShapeDtypeStruct((1, 1), jnp.float32),
                   jax.ShapeDtypeStruct((1, 1), jnp.float32)],
    )(hist, dsum)


def kernel(inputs, emb):
    x = jnp.transpose(inputs, (0, 2, 3, 1))
    flat = x.reshape(-1, DIM)
    x2 = jnp.sum(flat ** 2, axis=1, keepdims=True)
    e2 = jnp.sum(emb ** 2, axis=1)[None, :]
    xb = (flat * 2.0).astype(jnp.bfloat16)
    eb = emb.astype(jnp.bfloat16)
    idx2d, dsum = _dist_argmin(xb, eb, x2, e2)
    idx = idx2d.reshape(NTOK)

    gather_hist, nw = _make_sc_gather_hist()
    zeros = jnp.zeros((NUM_E,), jnp.float32)
    quant, hist = gather_hist(emb, idx.reshape(NTOK // 128, 128), zeros)

    loss2d, perp2d = _finish(hist, dsum)
    out_q = jnp.transpose(quant.reshape(8, 32, 32, DIM), (0, 3, 1, 2))
    return (loss2d[0, 0], out_q, perp2d[0, 0], idx.reshape(8, 32, 32))
